# baseline (speedup 1.0000x reference)
import jax
import jax.numpy as jnp
from jax.experimental import pallas as pl
from jax.experimental.pallas import tpu as pltpu

_N_CLUSTERS = 512
_D_MODEL = 768
_N_POINTS = 576
_KMEANS_ITERS = 10
_PAIR = 2


def _kmeans_kernel(x_ref, gamma_ref, beta_ref, centers_ref, soft_ref):
    gamma = gamma_ref[...][None, :]
    beta = beta_ref[...][None, :]

    def layer_norm(x):
        mu = jnp.mean(x, axis=-1, keepdims=True)
        xc = x - mu
        var = jnp.mean(xc * xc, axis=-1, keepdims=True)
        return xc * jax.lax.rsqrt(var + 1e-5) * gamma + beta

    xs = [layer_norm(x_ref[i]) for i in range(_PAIR)]
    xm2s = [x * -2.0 for x in xs]

    ones_row = jnp.ones((8, _N_POINTS), dtype=jnp.float32)

    def dists(xm2, centers):
        c2 = jnp.sum(centers * centers, axis=-1)
        prod = jax.lax.dot_general(xm2, centers, (((1,), (1,)), ((), ())),
                                   preferred_element_type=jnp.float32)
        return prod + c2[None, :]

    def step(x, xm2, centers):
        d = dists(xm2, centers)
        dmin = jnp.min(d, axis=-1, keepdims=True)
        onehot = (d == dmin).astype(jnp.float32)
        sums = jax.lax.dot_general(onehot, x, (((0,), (0,)), ((), ())),
                                   preferred_element_type=jnp.float32)
        counts = jax.lax.dot_general(ones_row, onehot, (((1,), (0,)), ((), ())),
                                     preferred_element_type=jnp.float32)[0]
        recip = 1.0 / jnp.maximum(counts, 1.0)
        new_centers = sums * recip[:, None]
        return jnp.where(counts[:, None] > 0, new_centers, centers)

    cs = [x[:_N_CLUSTERS] for x in xs]
    for _ in range(_KMEANS_ITERS):
        cs = [step(xs[i], xm2s[i], cs[i]) for i in range(_PAIR)]

    def finish(xm2, centers):
        d = dists(xm2, centers)
        m = jnp.max(-d, axis=-1, keepdims=True)
        e = jnp.exp(-d - m)
        return e / jnp.sum(e, axis=-1, keepdims=True)

    for i in range(_PAIR):
        centers_ref[i] = cs[i]
        soft_ref[i] = finish(xm2s[i], cs[i])


def kernel(patches, gamma, beta):
    B, N, D = patches.shape
    centers, soft = pl.pallas_call(
        _kmeans_kernel,
        grid=(B // _PAIR,),
        in_specs=[
            pl.BlockSpec((_PAIR, N, D), lambda b: (b, 0, 0)),
            pl.BlockSpec((D,), lambda b: (0,)),
            pl.BlockSpec((D,), lambda b: (0,)),
        ],
        out_specs=[
            pl.BlockSpec((_PAIR, _N_CLUSTERS, D), lambda b: (b, 0, 0)),
            pl.BlockSpec((_PAIR, N, _N_CLUSTERS), lambda b: (b, 0, 0)),
        ],
        out_shape=[
            jax.ShapeDtypeStruct((B, _N_CLUSTERS, D), jnp.float32),
            jax.ShapeDtypeStruct((B, N, _N_CLUSTERS), jnp.float32),
        ],
        compiler_params=pltpu.CompilerParams(
            dimension_semantics=("parallel",),
        ),
    )(patches, gamma, beta)
    return (centers, soft)


# speedup vs baseline: 1.0379x; 1.0379x over previous
import jax
import jax.numpy as jnp
from jax.experimental import pallas as pl
from jax.experimental.pallas import tpu as pltpu

_N_CLUSTERS = 512
_D_MODEL = 768
_N_POINTS = 576
_KMEANS_ITERS = 10


def _kmeans_kernel(x_ref, gamma_ref, beta_ref, centers_ref, soft_ref):
    x = x_ref[0]
    gamma = gamma_ref[...]
    beta = beta_ref[...]
    mu = jnp.mean(x, axis=-1, keepdims=True)
    xc = x - mu
    var = jnp.mean(xc * xc, axis=-1, keepdims=True)
    x = xc * jax.lax.rsqrt(var + 1e-5) * gamma[None, :] + beta[None, :]

    xm2 = x * -2.0
    col_ids = jax.lax.broadcasted_iota(jnp.int32, (_N_POINTS, _N_CLUSTERS), 1)
    ones_row = jnp.ones((8, _N_POINTS), dtype=jnp.float32)
    big = jnp.int32(1 << 30)

    def dists(centers):
        c2 = jnp.sum(centers * centers, axis=-1)
        prod = jax.lax.dot_general(xm2, centers, (((1,), (1,)), ((), ())),
                                   preferred_element_type=jnp.float32)
        return prod + c2[None, :]

    def body(_, centers):
        d = dists(centers)
        dmin = jnp.min(d, axis=-1, keepdims=True)
        onehot = (d == dmin).astype(jnp.float32)
        sums = jax.lax.dot_general(onehot, x, (((0,), (0,)), ((), ())),
                                   preferred_element_type=jnp.float32)
        counts = jax.lax.dot_general(ones_row, onehot, (((1,), (0,)), ((), ())),
                                     preferred_element_type=jnp.float32)[0]
        recip = 1.0 / jnp.maximum(counts, 1.0)
        new_centers = sums * recip[:, None]
        return jnp.where(counts[:, None] > 0, new_centers, centers)

    centers = x[:_N_CLUSTERS]
    for _ in range(_KMEANS_ITERS):
        centers = body(0, centers)
    d = dists(centers)
    centers_ref[0] = centers
    m = jnp.max(-d, axis=-1, keepdims=True)
    e = jnp.exp(-d - m)
    soft_ref[0] = e / jnp.sum(e, axis=-1, keepdims=True)


def kernel(patches, gamma, beta):
    B, N, D = patches.shape
    centers, soft = pl.pallas_call(
        _kmeans_kernel,
        grid=(B,),
        in_specs=[
            pl.BlockSpec((1, N, D), lambda b: (b, 0, 0)),
            pl.BlockSpec((D,), lambda b: (0,)),
            pl.BlockSpec((D,), lambda b: (0,)),
        ],
        out_specs=[
            pl.BlockSpec((1, _N_CLUSTERS, D), lambda b: (b, 0, 0)),
            pl.BlockSpec((1, N, _N_CLUSTERS), lambda b: (b, 0, 0)),
        ],
        out_shape=[
            jax.ShapeDtypeStruct((B, _N_CLUSTERS, D), jnp.float32),
            jax.ShapeDtypeStruct((B, N, _N_CLUSTERS), jnp.float32),
        ],
        compiler_params=pltpu.CompilerParams(
            dimension_semantics=("arbitrary",),
        ),
    )(patches, gamma, beta)
    return (centers, soft)


# speedup vs baseline: 1.0450x; 1.0068x over previous
import jax
import jax.numpy as jnp
from jax.experimental import pallas as pl
from jax.experimental.pallas import tpu as pltpu

_N_CLUSTERS = 512
_D_MODEL = 768
_N_POINTS = 576
_KMEANS_ITERS = 10


def _kmeans_kernel(x_ref, gamma_ref, beta_ref, centers_ref, soft_ref):
    x = x_ref[0]
    gamma = gamma_ref[...]
    beta = beta_ref[...]
    mu = jnp.mean(x, axis=-1, keepdims=True)
    xc = x - mu
    var = jnp.mean(xc * xc, axis=-1, keepdims=True)
    x = xc * jax.lax.rsqrt(var + 1e-5) * gamma[None, :] + beta[None, :]

    xm2 = x * -2.0
    x2 = jnp.sum(x * x, axis=-1, keepdims=True)
    ones_row = jnp.ones((8, _N_POINTS), dtype=jnp.float32)

    def dists(centers):
        c2 = jnp.sum(centers * centers, axis=-1)
        prod = jax.lax.dot_general(xm2, centers, (((1,), (1,)), ((), ())),
                                   preferred_element_type=jnp.float32)
        return (x2 + prod) + c2[None, :]

    def body(_, centers):
        d = dists(centers)
        dmin = jnp.min(d, axis=-1, keepdims=True)
        onehot = (d == dmin).astype(jnp.float32)
        sums = jax.lax.dot_general(onehot, x, (((0,), (0,)), ((), ())),
                                   preferred_element_type=jnp.float32)
        counts = jax.lax.dot_general(ones_row, onehot, (((1,), (0,)), ((), ())),
                                     preferred_element_type=jnp.float32)[0]
        new_centers = sums / jnp.maximum(counts, 1.0)[:, None]
        return jnp.where(counts[:, None] > 0, new_centers, centers)

    centers = x[:_N_CLUSTERS]
    for _ in range(_KMEANS_ITERS):
        centers = body(0, centers)
    d = dists(centers)
    centers_ref[0] = centers
    m = jnp.max(-d, axis=-1, keepdims=True)
    e = jnp.exp(-d - m)
    soft_ref[0] = e / jnp.sum(e, axis=-1, keepdims=True)


def kernel(patches, gamma, beta):
    B, N, D = patches.shape
    centers, soft = pl.pallas_call(
        _kmeans_kernel,
        grid=(B,),
        in_specs=[
            pl.BlockSpec((1, N, D), lambda b: (b, 0, 0)),
            pl.BlockSpec((D,), lambda b: (0,)),
            pl.BlockSpec((D,), lambda b: (0,)),
        ],
        out_specs=[
            pl.BlockSpec((1, _N_CLUSTERS, D), lambda b: (b, 0, 0)),
            pl.BlockSpec((1, N, _N_CLUSTERS), lambda b: (b, 0, 0)),
        ],
        out_shape=[
            jax.ShapeDtypeStruct((B, _N_CLUSTERS, D), jnp.float32),
            jax.ShapeDtypeStruct((B, N, _N_CLUSTERS), jnp.float32),
        ],
        compiler_params=pltpu.CompilerParams(
            dimension_semantics=("parallel",),
        ),
    )(patches, gamma, beta)
    return (centers, soft)
